# baseline (speedup 1.0000x reference)
import jax
import jax.numpy as jnp
from jax.experimental import pallas as pl
from jax.experimental.pallas import tpu as pltpu

IN_F = 1024
OUT_F = 1024
RANK = 16
NE = 16
SCALING = 2.0
TB = 512

_DN_T = (((1,), (1,)), ((), ()))
_DN_LT = (((0,), (0,)), ((), ()))


def _routing_weights(logits):
    m = jnp.max(logits, axis=-1, keepdims=True)
    e = jnp.exp(logits - m)
    iota = jax.lax.broadcasted_iota(jnp.int32, e.shape, 1)
    i1 = jnp.min(jnp.where(e == 1.0, iota, NE), axis=-1, keepdims=True)
    oh1 = iota == i1
    em = jnp.where(oh1, -1.0, e)
    m2 = jnp.max(em, axis=-1, keepdims=True)
    i2 = jnp.min(jnp.where(em == m2, iota, NE), axis=-1, keepdims=True)
    sel = oh1 | (iota == i2)
    return jnp.where(sel, e, 0.0) / (1.0 + m2)


def _eye(n, dtype):
    r = jax.lax.broadcasted_iota(jnp.int32, (n, n), 0)
    c = jax.lax.broadcasted_iota(jnp.int32, (n, n), 1)
    return (r == c).astype(dtype)


def _fused_kernel(x_ref, bw_ref, bb_ref, rw_ref, a_ref, b_ref, out_ref,
                  bwt_scr, at_scr, bf_scr, rwt_scr):
    @pl.when(pl.program_id(0) == 0)
    def _prep():
        ident = _eye(IN_F, jnp.bfloat16)
        bwt_scr[...] = jax.lax.dot_general(
            bw_ref[...].astype(jnp.bfloat16), ident, _DN_LT,
            preferred_element_type=jnp.float32).astype(jnp.bfloat16)
        at_scr[...] = jax.lax.dot_general(
            a_ref[...].astype(jnp.bfloat16), _eye(NE * RANK, jnp.bfloat16),
            _DN_LT, preferred_element_type=jnp.float32).astype(jnp.bfloat16)
        rwt_scr[...] = jax.lax.dot_general(
            rw_ref[...], _eye(NE, jnp.float32), _DN_LT,
            preferred_element_type=jnp.float32)
        b16 = (b_ref[...] * SCALING).astype(jnp.bfloat16)
        for ex in range(NE):
            bf_scr[ex * RANK:(ex + 1) * RANK, :] = jax.lax.dot_general(
                b16[ex], ident, _DN_LT,
                preferred_element_type=jnp.float32).astype(jnp.bfloat16)

    xb = x_ref[0]
    logits = jnp.dot(xb, rwt_scr[...], preferred_element_type=jnp.float32)
    w = _routing_weights(logits)
    xb16 = xb.astype(jnp.bfloat16)
    base = jnp.dot(xb16, bwt_scr[...], preferred_element_type=jnp.float32)
    h = jnp.dot(xb16, at_scr[...], preferred_element_type=jnp.float32)
    er = jax.lax.broadcasted_iota(jnp.int32, (NE, NE * RANK), 0)
    ec = jax.lax.broadcasted_iota(jnp.int32, (NE, NE * RANK), 1)
    expand = (ec // RANK == er).astype(jnp.float32)
    hw = (h * jnp.dot(w, expand,
                      preferred_element_type=jnp.float32)).astype(jnp.bfloat16)
    lora = jnp.dot(hw, bf_scr[...], preferred_element_type=jnp.float32)
    out_ref[0] = base + bb_ref[...] + lora


def kernel(x, base_W, base_b, router_W, lora_A, lora_B):
    orig_shape = x.shape
    n_tok = orig_shape[0] * orig_shape[1]
    x3 = x.reshape(1, n_tok, IN_F)
    grid = (n_tok // TB,)

    out = pl.pallas_call(
        _fused_kernel,
        grid=grid,
        in_specs=[
            pl.BlockSpec((1, TB, IN_F), lambda i: (0, i, 0)),
            pl.BlockSpec((OUT_F, IN_F), lambda i: (0, 0)),
            pl.BlockSpec((1, OUT_F), lambda i: (0, 0)),
            pl.BlockSpec((NE, IN_F), lambda i: (0, 0)),
            pl.BlockSpec((NE * RANK, IN_F), lambda i: (0, 0)),
            pl.BlockSpec((NE, OUT_F, RANK), lambda i: (0, 0, 0)),
        ],
        out_specs=pl.BlockSpec((1, TB, OUT_F), lambda i: (0, i, 0)),
        out_shape=jax.ShapeDtypeStruct((1, n_tok, OUT_F), x.dtype),
        scratch_shapes=[
            pltpu.VMEM((IN_F, OUT_F), jnp.bfloat16),
            pltpu.VMEM((IN_F, NE * RANK), jnp.bfloat16),
            pltpu.VMEM((NE * RANK, OUT_F), jnp.bfloat16),
            pltpu.VMEM((IN_F, NE), jnp.float32),
        ],
        compiler_params=pltpu.CompilerParams(
            dimension_semantics=("arbitrary",),
        ),
    )(x3, base_W, base_b.reshape(1, OUT_F), router_W,
      lora_A.reshape(NE * RANK, IN_F), lora_B)
    return out.reshape(*orig_shape[:-1], OUT_F)


# speedup vs baseline: 1.0914x; 1.0914x over previous
import jax
import jax.numpy as jnp
from jax.experimental import pallas as pl
from jax.experimental.pallas import tpu as pltpu

IN_F = 1024
OUT_F = 1024
RANK = 16
NE = 16
SCALING = 2.0
TB = 512

_DN_T = (((1,), (1,)), ((), ()))
_DN_LT = (((0,), (0,)), ((), ()))


def _routing_weights(logits):
    m = jnp.max(logits, axis=-1, keepdims=True)
    e = jnp.exp(logits - m)
    iota = jax.lax.broadcasted_iota(jnp.int32, e.shape, 1)
    i1 = jnp.min(jnp.where(e == 1.0, iota, NE), axis=-1, keepdims=True)
    oh1 = iota == i1
    em = jnp.where(oh1, -1.0, e)
    m2 = jnp.max(em, axis=-1, keepdims=True)
    i2 = jnp.min(jnp.where(em == m2, iota, NE), axis=-1, keepdims=True)
    sel = oh1 | (iota == i2)
    return jnp.where(sel, e, 0.0) / (1.0 + m2)


def _eye(n, dtype):
    r = jax.lax.broadcasted_iota(jnp.int32, (n, n), 0)
    c = jax.lax.broadcasted_iota(jnp.int32, (n, n), 1)
    return (r == c).astype(dtype)


def _fused_kernel(x_ref, bw_ref, bb_ref, rw_ref, a_ref, b_ref, out_ref,
                  bw_scr, a_scr, bf_scr):
    @pl.when(pl.program_id(0) == 0)
    def _prep():
        bw_scr[...] = bw_ref[...].astype(jnp.bfloat16)
        a_scr[...] = a_ref[...].astype(jnp.bfloat16)
        ident = _eye(IN_F, jnp.bfloat16)
        b16 = (b_ref[...] * SCALING).astype(jnp.bfloat16)
        for ex in range(NE):
            bf_scr[ex * RANK:(ex + 1) * RANK, :] = jax.lax.dot_general(
                b16[ex], ident, _DN_LT,
                preferred_element_type=jnp.float32).astype(jnp.bfloat16)

    xb = x_ref[0]
    logits = jax.lax.dot_general(xb, rw_ref[...], _DN_T,
                                 preferred_element_type=jnp.float32)
    w = _routing_weights(logits)
    xb16 = xb.astype(jnp.bfloat16)
    base = jax.lax.dot_general(xb16, bw_scr[...], _DN_T,
                               preferred_element_type=jnp.float32)
    h = jax.lax.dot_general(xb16, a_scr[...], _DN_T,
                            preferred_element_type=jnp.float32)
    er = jax.lax.broadcasted_iota(jnp.int32, (NE, NE * RANK), 0)
    ec = jax.lax.broadcasted_iota(jnp.int32, (NE, NE * RANK), 1)
    expand = (ec // RANK == er).astype(jnp.float32)
    hw = (h * jnp.dot(w, expand,
                      preferred_element_type=jnp.float32)).astype(jnp.bfloat16)
    lora = jnp.dot(hw, bf_scr[...], preferred_element_type=jnp.float32)
    out_ref[0] = base + bb_ref[...] + lora


def kernel(x, base_W, base_b, router_W, lora_A, lora_B):
    orig_shape = x.shape
    n_tok = orig_shape[0] * orig_shape[1]
    x3 = x.reshape(1, n_tok, IN_F)
    grid = (n_tok // TB,)

    out = pl.pallas_call(
        _fused_kernel,
        grid=grid,
        in_specs=[
            pl.BlockSpec((1, TB, IN_F), lambda i: (0, i, 0)),
            pl.BlockSpec((OUT_F, IN_F), lambda i: (0, 0)),
            pl.BlockSpec((1, OUT_F), lambda i: (0, 0)),
            pl.BlockSpec((NE, IN_F), lambda i: (0, 0)),
            pl.BlockSpec((NE * RANK, IN_F), lambda i: (0, 0)),
            pl.BlockSpec((NE, OUT_F, RANK), lambda i: (0, 0, 0)),
        ],
        out_specs=pl.BlockSpec((1, TB, OUT_F), lambda i: (0, i, 0)),
        out_shape=jax.ShapeDtypeStruct((1, n_tok, OUT_F), x.dtype),
        scratch_shapes=[
            pltpu.VMEM((OUT_F, IN_F), jnp.bfloat16),
            pltpu.VMEM((NE * RANK, IN_F), jnp.bfloat16),
            pltpu.VMEM((NE * RANK, OUT_F), jnp.bfloat16),
        ],
        compiler_params=pltpu.CompilerParams(
            dimension_semantics=("arbitrary",),
        ),
    )(x3, base_W, base_b.reshape(1, OUT_F), router_W,
      lora_A.reshape(NE * RANK, IN_F), lora_B)
    return out.reshape(*orig_shape[:-1], OUT_F)
